# baseline (speedup 1.0000x reference)
import functools

import numpy as np

import jax
import jax.numpy as jnp
from jax import lax
from jax.experimental import pallas as pl
from jax.experimental.pallas import tpu as pltpu
from jax.experimental.pallas import tpu_sc as plsc

_NUM_SC = 2
_NUM_SUBCORES = 16
_NUM_WORKERS = _NUM_SC * _NUM_SUBCORES
_LANES = 16


def _embed_perm(embed: int) -> np.ndarray:
    m = np.arange(embed // 2)
    lo = 32 * (m // 16) + (m % 16)
    return np.concatenate([lo, lo + 16]).astype(np.int32)


def _make_sc_gather_pack(n_tokens: int, seq: int, embed: int, chunk: int):
    per_worker = n_tokens // _NUM_WORKERS
    assert per_worker % chunk == 0 and seq % per_worker == 0
    n_chunks = per_worker // chunk
    half = embed // 2
    groups = embed // 32
    mesh = plsc.VectorSubcoreMesh(core_axis_name="c", subcore_axis_name="s")

    @functools.partial(
        pl.kernel,
        mesh=mesh,
        out_type=jax.ShapeDtypeStruct((n_tokens, half), jnp.int32),
        scratch_types=[
            pltpu.VMEM((per_worker,), jnp.int32),
            pltpu.VMEM((chunk, embed), jnp.float32),
            pltpu.VMEM((chunk, embed), jnp.float32),
            pltpu.VMEM((chunk, embed), jnp.float32),
            pltpu.VMEM((chunk, embed), jnp.float32),
            pltpu.VMEM((chunk, half), jnp.int32),
            pltpu.VMEM((chunk, half), jnp.int32),
            pltpu.SemaphoreType.DMA,
            pltpu.SemaphoreType.DMA,
            pltpu.SemaphoreType.DMA,
            pltpu.SemaphoreType.DMA,
            pltpu.SemaphoreType.DMA,
            pltpu.SemaphoreType.DMA,
        ],
    )
    def gather(table_hbm, pos_hbm, idx_hbm, out_hbm,
               idx_v, w_a, w_b, p_a, p_b, o_a, o_b,
               gw_a, gw_b, gp_a, gp_b, os_a, os_b):
        wid = lax.axis_index("s") * _NUM_SC + lax.axis_index("c")
        base = wid * per_worker
        pos_base = lax.rem(base, seq)
        wbufs, pbufs, obufs = (w_a, w_b), (p_a, p_b), (o_a, o_b)
        gwsems, gpsems, osems = (gw_a, gw_b), (gp_a, gp_b), (os_a, os_b)

        pltpu.sync_copy(idx_hbm.at[pl.ds(base, per_worker)], idx_v)

        def start_word(c):
            return pltpu.async_copy(
                table_hbm.at[idx_v.at[pl.ds(c * chunk, chunk)]],
                wbufs[c % 2], gwsems[c % 2])

        def start_pos(c):
            return pltpu.async_copy(
                pos_hbm.at[pl.ds(pos_base + c * chunk, chunk)],
                pbufs[c % 2], gpsems[c % 2])

        def start_out(c):
            return pltpu.async_copy(
                obufs[c % 2],
                out_hbm.at[pl.ds(base + c * chunk, chunk)], osems[c % 2])

        def convert(wbuf, pbuf, obuf):
            hi_mask = jnp.int32(-65536)

            def row_body(r, _):
                for g in range(groups):
                    xa = wbuf[r, pl.ds(32 * g, 16)] + pbuf[r, pl.ds(32 * g, 16)]
                    xb = (wbuf[r, pl.ds(32 * g + 16, 16)]
                          + pbuf[r, pl.ds(32 * g + 16, 16)])
                    ua = lax.bitcast_convert_type(xa, jnp.int32)
                    ub = lax.bitcast_convert_type(xb, jnp.int32)
                    packed = lax.shift_right_logical(ua, 16) | (ub & hi_mask)
                    obuf[r, pl.ds(16 * g, 16)] = packed
                return 0

            lax.fori_loop(0, chunk, row_body, 0)

        words = [None] * n_chunks
        poss = [None] * n_chunks
        outs = [None] * n_chunks
        words[0] = start_word(0)
        poss[0] = start_pos(0)
        for c in range(n_chunks):
            b = c % 2
            words[c].wait()
            poss[c].wait()
            if c + 1 < n_chunks:
                words[c + 1] = start_word(c + 1)
                poss[c + 1] = start_pos(c + 1)
            if c >= 2:
                outs[c - 2].wait()
            convert(wbufs[b], pbufs[b], obufs[b])
            outs[c] = start_out(c)
        outs[n_chunks - 2].wait()
        outs[n_chunks - 1].wait()

    return gather


def _ln_matmul_body(x_ref, g_ref, bt_ref, w_ref, bias_ref, o_ref):
    xi = x_ref[...]
    a = lax.bitcast_convert_type(lax.shift_left(xi, 16), jnp.float32)
    bb = lax.bitcast_convert_type(xi & jnp.int32(-65536), jnp.float32)
    x = jnp.concatenate([a, bb], axis=1)
    mu = jnp.mean(x, axis=-1, keepdims=True)
    xc = x - mu
    var = jnp.mean(xc * xc, axis=-1, keepdims=True)
    xn = xc * lax.rsqrt(var + 1e-12)
    xn = xn * g_ref[...] + bt_ref[...]
    o_ref[...] = (
        jnp.dot(
            xn.astype(jnp.bfloat16),
            w_ref[...],
            preferred_element_type=jnp.float32,
        )
        + bias_ref[...]
    )


def _make_tc_fused(n_tokens: int, embed: int, hidden: int, tm: int):
    half = embed // 2
    grid = (n_tokens // tm,)

    return pl.pallas_call(
        _ln_matmul_body,
        grid=grid,
        in_specs=[
            pl.BlockSpec((tm, half), lambda i: (i, 0)),
            pl.BlockSpec((1, embed), lambda i: (0, 0)),
            pl.BlockSpec((1, embed), lambda i: (0, 0)),
            pl.BlockSpec((embed, hidden), lambda i: (0, 0)),
            pl.BlockSpec((1, hidden), lambda i: (0, 0)),
        ],
        out_specs=pl.BlockSpec((tm, hidden), lambda i: (i, 0)),
        out_shape=jax.ShapeDtypeStruct((n_tokens, hidden), jnp.float32),
    )


def kernel(input_ids, word_table, pos_table, ln_gamma, ln_beta, W, b):
    bsz, seq = input_ids.shape
    vocab, embed = word_table.shape
    hidden = W.shape[1]
    n_tokens = bsz * seq

    ids_flat = input_ids.reshape(n_tokens).astype(jnp.int32)
    packed = _make_sc_gather_pack(n_tokens, seq, embed, chunk=16)(
        word_table, pos_table[:seq], ids_flat)

    perm = jnp.asarray(_embed_perm(embed))
    fused = _make_tc_fused(n_tokens, embed, hidden, tm=1024)
    out = fused(
        packed,
        ln_gamma[perm].reshape(1, embed),
        ln_beta[perm].reshape(1, embed),
        W[perm].astype(jnp.bfloat16),
        b.reshape(1, hidden),
    )
    return out.reshape(bsz, seq, hidden)


# speedup vs baseline: 1.4556x; 1.4556x over previous
import functools

import jax
import jax.numpy as jnp
from jax import lax
from jax.experimental import pallas as pl
from jax.experimental.pallas import tpu as pltpu
from jax.experimental.pallas import tpu_sc as plsc

_NUM_SC = 2
_NUM_SUBCORES = 16
_NUM_WORKERS = _NUM_SC * _NUM_SUBCORES


def _make_sc_gather(n_tokens: int, embed: int, chunk: int):
    per_worker = n_tokens // _NUM_WORKERS
    assert per_worker % chunk == 0
    n_chunks = per_worker // chunk
    mesh = plsc.VectorSubcoreMesh(core_axis_name="c", subcore_axis_name="s")

    @functools.partial(
        pl.kernel,
        mesh=mesh,
        out_type=jax.ShapeDtypeStruct((n_tokens, embed), jnp.float32),
        scratch_types=[
            pltpu.VMEM((per_worker,), jnp.int32),
            pltpu.VMEM((chunk, embed), jnp.float32),
            pltpu.VMEM((chunk, embed), jnp.float32),
            pltpu.SemaphoreType.DMA,
            pltpu.SemaphoreType.DMA,
            pltpu.SemaphoreType.DMA,
            pltpu.SemaphoreType.DMA,
        ],
    )
    def gather(table_hbm, idx_hbm, out_hbm, idx_v, rows_a, rows_b,
               gsem_a, gsem_b, osem_a, osem_b):
        wid = lax.axis_index("s") * _NUM_SC + lax.axis_index("c")
        base = wid * per_worker
        bufs = (rows_a, rows_b)
        gsems = (gsem_a, gsem_b)
        osems = (osem_a, osem_b)
        pltpu.sync_copy(idx_hbm.at[pl.ds(base, per_worker)], idx_v)

        def start_gather(c):
            return pltpu.async_copy(
                table_hbm.at[idx_v.at[pl.ds(c * chunk, chunk)]],
                bufs[c % 2],
                gsems[c % 2],
            )

        def start_out(c):
            return pltpu.async_copy(
                bufs[c % 2],
                out_hbm.at[pl.ds(base + c * chunk, chunk)],
                osems[c % 2],
            )

        gathers = [None] * n_chunks
        outs = [None] * n_chunks
        gathers[0] = start_gather(0)
        for c in range(n_chunks):
            gathers[c].wait()
            outs[c] = start_out(c)
            if c + 1 < n_chunks:
                if c >= 1:
                    outs[c - 1].wait()
                gathers[c + 1] = start_gather(c + 1)
        outs[n_chunks - 2].wait()
        outs[n_chunks - 1].wait()

    return gather


def _ln_matmul_body(x_ref, pos_ref, g_ref, bt_ref, w_ref, bias_ref, o_ref):
    x = x_ref[...] + pos_ref[...]
    mu = jnp.mean(x, axis=-1, keepdims=True)
    xc = x - mu
    var = jnp.mean(xc * xc, axis=-1, keepdims=True)
    xn = xc * lax.rsqrt(var + 1e-12)
    xn = xn * g_ref[...] + bt_ref[...]
    o_ref[...] = (
        jnp.dot(
            xn.astype(jnp.bfloat16),
            w_ref[...],
            preferred_element_type=jnp.float32,
        )
        + bias_ref[...]
    )


def _make_tc_fused(n_tokens: int, seq: int, embed: int, hidden: int, tm: int):
    n_batch = n_tokens // seq
    pos_blocks = seq // tm
    grid = (pos_blocks, n_batch)

    return pl.pallas_call(
        _ln_matmul_body,
        grid=grid,
        in_specs=[
            pl.BlockSpec((tm, embed), lambda p, j: (j * pos_blocks + p, 0)),
            pl.BlockSpec((tm, embed), lambda p, j: (p, 0)),
            pl.BlockSpec((1, embed), lambda p, j: (0, 0)),
            pl.BlockSpec((1, embed), lambda p, j: (0, 0)),
            pl.BlockSpec((embed, hidden), lambda p, j: (0, 0)),
            pl.BlockSpec((1, hidden), lambda p, j: (0, 0)),
        ],
        out_specs=pl.BlockSpec((tm, hidden), lambda p, j: (j * pos_blocks + p, 0)),
        out_shape=jax.ShapeDtypeStruct((n_tokens, hidden), jnp.float32),
    )


def kernel(input_ids, word_table, pos_table, ln_gamma, ln_beta, W, b):
    bsz, seq = input_ids.shape
    vocab, embed = word_table.shape
    hidden = W.shape[1]
    n_tokens = bsz * seq

    ids_flat = input_ids.reshape(n_tokens).astype(jnp.int32)

    gathered = _make_sc_gather(n_tokens, embed, chunk=32)(word_table, ids_flat)
    fused = _make_tc_fused(n_tokens, seq, embed, hidden, tm=1024)
    out = fused(
        gathered,
        pos_table[:seq],
        ln_gamma.reshape(1, embed),
        ln_beta.reshape(1, embed),
        W.astype(jnp.bfloat16),
        b.reshape(1, hidden),
    )
    return out.reshape(bsz, seq, hidden)
